# Initial kernel scaffold; baseline (speedup 1.0000x reference)
import jax
import jax.numpy as jnp
from jax.experimental import pallas as pl


def kernel(entity_vec, node_reprs, edge_src, edge_dst, edge_rel, edge_conf, W_msg, b_msg, W_self, b_self):
    raise NotImplementedError("write your pallas kernel here")


# speedup vs baseline: 1.5276x; 1.5276x over previous
import functools

import jax
import jax.numpy as jnp
from jax import lax
from jax.experimental import pallas as pl
from jax.experimental.pallas import tpu as pltpu
from jax.experimental.pallas import tpu_sc as plsc

N_NODES = 10000
N_EDGES = 320000
HID = 128
BATCH = 32

NC = 2
NS = 16
NW = NC * NS
EPW = N_EDGES // NW
CHUNK = 80
NCHUNK = EPW // CHUNK
ROWS_PT = 624
ROWS_TAIL = N_NODES - ROWS_PT * NS
LANES = 16
GROUPS = HID // LANES

f32 = jnp.float32


def _edge_body(p_hbm, r_hbm, src_hbm, dst_hbm, conf_hbm, zero_hbm, out_hbm,
               src_v, dst_v, conf_v, r_v, g_v, agg_sp, sem):
    c = lax.axis_index("c")
    s = lax.axis_index("s")
    wid = c * NS + s

    pltpu.sync_copy(zero_hbm.at[pl.ds(s * ROWS_PT, ROWS_PT)],
                    agg_sp.at[pl.ds(s * ROWS_PT, ROWS_PT)])

    @pl.when(s == NS - 1)
    def _():
        pltpu.sync_copy(zero_hbm.at[pl.ds(ROWS_PT * NS, ROWS_TAIL)],
                        agg_sp.at[pl.ds(ROWS_PT * NS, ROWS_TAIL)])

    plsc.subcore_barrier()

    base = wid * EPW

    def chunk_body(j, carry):
        off = base + j * CHUNK
        pltpu.sync_copy(src_hbm.at[pl.ds(off, CHUNK)], src_v)
        pltpu.sync_copy(dst_hbm.at[pl.ds(off, CHUNK)], dst_v)
        pltpu.sync_copy(conf_hbm.at[pl.ds(off, CHUNK)], conf_v)
        pltpu.sync_copy(r_hbm.at[pl.ds(off, CHUNK)], r_v)
        pltpu.async_copy(p_hbm.at[src_v], g_v, sem).wait()

        def grp_body(g, carry2):
            cf16 = conf_v[pl.ds(g * LANES, LANES)]
            for i in range(LANES):
                cf = cf16[i]
                e = g * LANES + i
                for h in range(GROUPS):
                    sl = pl.ds(h * LANES, LANES)
                    x = g_v[e, sl] + r_v[e, sl]
                    ex = jnp.exp(x + x)
                    g_v[e, sl] = (1.0 - 2.0 / (ex + 1.0)) * cf
            return carry2

        lax.fori_loop(0, CHUNK // LANES, grp_body, 0, unroll=False)
        pltpu.sync_copy(g_v, agg_sp.at[dst_v], add=True)
        return carry

    lax.fori_loop(0, NCHUNK, chunk_body, 0, unroll=False)
    plsc.subcore_barrier()
    pltpu.sync_copy(agg_sp.at[pl.ds(s * ROWS_PT, ROWS_PT)],
                    out_hbm.at[c, pl.ds(s * ROWS_PT, ROWS_PT)])

    @pl.when(s == NS - 1)
    def _():
        pltpu.sync_copy(agg_sp.at[pl.ds(ROWS_PT * NS, ROWS_TAIL)],
                        out_hbm.at[c, pl.ds(ROWS_PT * NS, ROWS_TAIL)])


@functools.cache
def _edge_pass():
    return functools.partial(
        pl.kernel,
        out_type=jax.ShapeDtypeStruct((NC, N_NODES, HID), f32),
        mesh=plsc.VectorSubcoreMesh(core_axis_name="c", subcore_axis_name="s",
                                    num_cores=NC, num_subcores=NS),
        scratch_types=[
            pltpu.VMEM((CHUNK,), jnp.int32),
            pltpu.VMEM((CHUNK,), jnp.int32),
            pltpu.VMEM((CHUNK,), f32),
            pltpu.VMEM((CHUNK, HID), f32),
            pltpu.VMEM((CHUNK, HID), f32),
            pltpu.VMEM_SHARED((N_NODES, HID), f32),
            pltpu.SemaphoreType.DMA,
        ],
    )(_edge_body)


def _dot(x, w):
    return jnp.dot(x, w, preferred_element_type=f32,
                   precision=lax.Precision.HIGHEST)


def _relproj_body(rel_ref, wb_ref, b_ref, out_ref):
    out_ref[...] = _dot(rel_ref[...], wb_ref[...]) + b_ref[...]


BE = 2000

_relproj = pl.pallas_call(
    _relproj_body,
    grid=(N_EDGES // BE,),
    in_specs=[
        pl.BlockSpec((BE, HID), lambda i: (i, 0)),
        pl.BlockSpec((HID, HID), lambda i: (0, 0)),
        pl.BlockSpec((1, HID), lambda i: (0, 0)),
    ],
    out_specs=pl.BlockSpec((BE, HID), lambda i: (i, 0)),
    out_shape=jax.ShapeDtypeStruct((N_EDGES, HID), f32),
)

BN = 2000


def _ntrans_body(ns_ref, wa_ref, wst_ref, bs_ref, p_ref, s_ref):
    x = ns_ref[...]
    p_ref[...] = _dot(x, wa_ref[...])
    s_ref[...] = _dot(x, wst_ref[...]) + bs_ref[...]


_ntrans = pl.pallas_call(
    _ntrans_body,
    grid=(N_NODES // BN,),
    in_specs=[
        pl.BlockSpec((BN, HID), lambda i: (i, 0)),
        pl.BlockSpec((HID, HID), lambda i: (0, 0)),
        pl.BlockSpec((HID, HID), lambda i: (0, 0)),
        pl.BlockSpec((1, HID), lambda i: (0, 0)),
    ],
    out_specs=[
        pl.BlockSpec((BN, HID), lambda i: (i, 0)),
        pl.BlockSpec((BN, HID), lambda i: (i, 0)),
    ],
    out_shape=[
        jax.ShapeDtypeStruct((N_NODES, HID), f32),
        jax.ShapeDtypeStruct((N_NODES, HID), f32),
    ],
)


def _update_body(agg_ref, sp_ref, wa_ref, wst_ref, bs_ref, p_ref, s_ref):
    x = jnp.tanh(sp_ref[...] + agg_ref[0] + agg_ref[1])
    p_ref[...] = _dot(x, wa_ref[...])
    s_ref[...] = _dot(x, wst_ref[...]) + bs_ref[...]


_update = pl.pallas_call(
    _update_body,
    grid=(N_NODES // BN,),
    in_specs=[
        pl.BlockSpec((NC, BN, HID), lambda i: (0, i, 0)),
        pl.BlockSpec((BN, HID), lambda i: (i, 0)),
        pl.BlockSpec((HID, HID), lambda i: (0, 0)),
        pl.BlockSpec((HID, HID), lambda i: (0, 0)),
        pl.BlockSpec((1, HID), lambda i: (0, 0)),
    ],
    out_specs=[
        pl.BlockSpec((BN, HID), lambda i: (i, 0)),
        pl.BlockSpec((BN, HID), lambda i: (i, 0)),
    ],
    out_shape=[
        jax.ShapeDtypeStruct((N_NODES, HID), f32),
        jax.ShapeDtypeStruct((N_NODES, HID), f32),
    ],
)


def _final_body(agg_ref, sp_ref, ent_ref, out_ref):
    ns2 = jnp.tanh(sp_ref[...] + agg_ref[0] + agg_ref[1])
    v = jnp.mean(ent_ref[...], axis=0, keepdims=True)
    vn = jnp.sqrt(jnp.sum(v * v))
    vs = v / jnp.maximum(vn, 1e-12)
    rn = jnp.sqrt(jnp.sum(ns2 * ns2, axis=1, keepdims=True))
    nn = ns2 / jnp.maximum(rn, 1e-12)
    sims = _dot(nn, vs.T)
    m = jnp.max(sims)
    idx = lax.broadcasted_iota(jnp.int32, (N_NODES, 1), 0)
    best = jnp.min(jnp.where(sims >= m, idx, N_NODES))
    ctx = jnp.sum(jnp.where(idx == best, ns2, 0.0), axis=0, keepdims=True)
    out_ref[...] = jnp.broadcast_to(ctx, (BATCH, HID))


_final = pl.pallas_call(
    _final_body,
    in_specs=[
        pl.BlockSpec((NC, N_NODES, HID), lambda: (0, 0, 0)),
        pl.BlockSpec((N_NODES, HID), lambda: (0, 0)),
        pl.BlockSpec((BATCH, HID), lambda: (0, 0)),
    ],
    out_specs=pl.BlockSpec((BATCH, HID), lambda: (0, 0)),
    out_shape=jax.ShapeDtypeStruct((BATCH, HID), f32),
)


def kernel(entity_vec, node_reprs, edge_src, edge_dst, edge_rel, edge_conf,
           W_msg, b_msg, W_self, b_self):
    wa = W_msg[:, :HID].T
    wb = W_msg[:, HID:].T
    wst = W_self.T
    bm = b_msg.reshape(1, HID)
    bs = b_self.reshape(1, HID)
    conf = edge_conf.reshape(N_EDGES)
    zeros = jnp.zeros((N_NODES, HID), f32)

    r = _relproj(edge_rel, wb, bm)
    p0, s0 = _ntrans(node_reprs, wa, wst, bs)
    edge_pass = _edge_pass()
    agg0 = edge_pass(p0, r, edge_src, edge_dst, conf, zeros)
    p1, s1 = _update(agg0, s0, wa, wst, bs)
    agg1 = edge_pass(p1, r, edge_src, edge_dst, conf, zeros)
    return _final(agg1, s1, entity_vec)


# speedup vs baseline: 2.0351x; 1.3322x over previous
import functools

import jax
import jax.numpy as jnp
from jax import lax
from jax.experimental import pallas as pl
from jax.experimental.pallas import tpu as pltpu
from jax.experimental.pallas import tpu_sc as plsc

N_NODES = 10000
N_EDGES = 320000
HID = 128
BATCH = 32

NC = 2
NS = 16
NW = NC * NS
EPW = N_EDGES // NW
CHUNK = 80
NCHUNK = EPW // CHUNK
ROWS_PT = 624
ROWS_TAIL = N_NODES - ROWS_PT * NS
LANES = 16
GROUPS = HID // LANES

f32 = jnp.float32


def _edge_body(p_hbm, r_hbm, src_hbm, dst_hbm, conf_hbm, zero_hbm, out_hbm,
               src0, dst0, conf0, r0, g0, src1, dst1, conf1, r1, g1, agg_sp,
               sem_l0, sem_g0, sem_l1, sem_g1):
    c = lax.axis_index("c")
    s = lax.axis_index("s")
    wid = c * NS + s
    base = wid * EPW

    pltpu.sync_copy(zero_hbm.at[pl.ds(s * ROWS_PT, ROWS_PT)],
                    agg_sp.at[pl.ds(s * ROWS_PT, ROWS_PT)])

    @pl.when(s == NS - 1)
    def _():
        pltpu.sync_copy(zero_hbm.at[pl.ds(ROWS_PT * NS, ROWS_TAIL)],
                        agg_sp.at[pl.ds(ROWS_PT * NS, ROWS_TAIL)])

    plsc.subcore_barrier()

    bufs = ((src0, dst0, conf0, r0, g0, sem_l0, sem_g0),
            (src1, dst1, conf1, r1, g1, sem_l1, sem_g1))

    def l_descs(j, b):
        src_b, dst_b, conf_b, r_b, _, sem_l, _ = bufs[b]
        off = base + j * CHUNK
        return (
            (src_hbm.at[pl.ds(off, CHUNK)], src_b, sem_l),
            (dst_hbm.at[pl.ds(off, CHUNK)], dst_b, sem_l),
            (conf_hbm.at[pl.ds(off, CHUNK)], conf_b, sem_l),
            (r_hbm.at[pl.ds(off, CHUNK)], r_b, sem_l),
        )

    def l_start(j, b):
        for d in l_descs(j, b):
            pltpu.async_copy(*d)

    def l_drain(j, b):
        for d in l_descs(j, b):
            pltpu.make_async_copy(*d).wait()

    def g_start(j, b):
        src_b, _, _, _, g_b, _, sem_g = bufs[b]
        pltpu.async_copy(p_hbm.at[src_b], g_b, sem_g)

    def g_drain(j, b):
        src_b, _, _, _, g_b, _, sem_g = bufs[b]
        pltpu.make_async_copy(p_hbm.at[src_b], g_b, sem_g).wait()

    def compute(j, b):
        _, dst_b, conf_b, r_b, g_b, _, _ = bufs[b]

        def grp_body(g, carry):
            cf16 = conf_b[pl.ds(g * LANES, LANES)]
            cf2_16 = cf16 + cf16
            for i in range(LANES):
                cf = cf16[i]
                cf2 = cf2_16[i]
                e = g * LANES + i
                for h in range(GROUPS):
                    sl = pl.ds(h * LANES, LANES)
                    t = g_b[e, sl] + r_b[e, sl]
                    g_b[e, sl] = cf - cf2 / (jnp.exp(t) + 1.0)
            return carry

        lax.fori_loop(0, CHUNK // LANES, grp_body, 0, unroll=False)
        pltpu.sync_copy(g_b, agg_sp.at[dst_b], add=True)

    l_start(0, 0)
    l_drain(0, 0)
    g_start(0, 0)
    l_start(1, 1)

    def pair_body(t, carry):
        j0 = 2 * t
        j1 = j0 + 1
        l_drain(j1, 1)
        g_start(j1, 1)
        g_drain(j0, 0)
        compute(j0, 0)
        l_start(j0 + 2, 0)

        g_drain(j1, 1)
        compute(j1, 1)

        @pl.when(j0 + 3 < NCHUNK)
        def _():
            l_start(j0 + 3, 1)

        l_drain(j0 + 2, 0)
        g_start(j0 + 2, 0)
        return carry

    lax.fori_loop(0, NCHUNK // 2, pair_body, 0, unroll=False)
    if NCHUNK % 2:
        g_drain(NCHUNK - 1, 0)
        compute(NCHUNK - 1, 0)

    plsc.subcore_barrier()
    pltpu.sync_copy(agg_sp.at[pl.ds(s * ROWS_PT, ROWS_PT)],
                    out_hbm.at[c, pl.ds(s * ROWS_PT, ROWS_PT)])

    @pl.when(s == NS - 1)
    def _():
        pltpu.sync_copy(agg_sp.at[pl.ds(ROWS_PT * NS, ROWS_TAIL)],
                        out_hbm.at[c, pl.ds(ROWS_PT * NS, ROWS_TAIL)])


@functools.cache
def _edge_pass():
    return functools.partial(
        pl.kernel,
        out_type=jax.ShapeDtypeStruct((NC, N_NODES, HID), f32),
        mesh=plsc.VectorSubcoreMesh(core_axis_name="c", subcore_axis_name="s",
                                    num_cores=NC, num_subcores=NS),
        scratch_types=[
            pltpu.VMEM((CHUNK,), jnp.int32),
            pltpu.VMEM((CHUNK,), jnp.int32),
            pltpu.VMEM((CHUNK,), f32),
            pltpu.VMEM((CHUNK, HID), f32),
            pltpu.VMEM((CHUNK, HID), f32),
            pltpu.VMEM((CHUNK,), jnp.int32),
            pltpu.VMEM((CHUNK,), jnp.int32),
            pltpu.VMEM((CHUNK,), f32),
            pltpu.VMEM((CHUNK, HID), f32),
            pltpu.VMEM((CHUNK, HID), f32),
            pltpu.VMEM_SHARED((N_NODES, HID), f32),
            pltpu.SemaphoreType.DMA,
            pltpu.SemaphoreType.DMA,
            pltpu.SemaphoreType.DMA,
            pltpu.SemaphoreType.DMA,
        ],
    )(_edge_body)


def _dot(x, w):
    return jnp.dot(x, w, preferred_element_type=f32,
                   precision=lax.Precision.HIGHEST)


def _relproj_body(rel_ref, wb_ref, b_ref, out_ref):
    out_ref[...] = _dot(rel_ref[...], wb_ref[...]) + b_ref[...]


BE = 2000

_relproj = pl.pallas_call(
    _relproj_body,
    grid=(N_EDGES // BE,),
    in_specs=[
        pl.BlockSpec((BE, HID), lambda i: (i, 0)),
        pl.BlockSpec((HID, HID), lambda i: (0, 0)),
        pl.BlockSpec((1, HID), lambda i: (0, 0)),
    ],
    out_specs=pl.BlockSpec((BE, HID), lambda i: (i, 0)),
    out_shape=jax.ShapeDtypeStruct((N_EDGES, HID), f32),
)

BN = 2000


def _ntrans_body(ns_ref, wa_ref, wst_ref, bs_ref, p_ref, s_ref):
    x = ns_ref[...]
    p_ref[...] = _dot(x, wa_ref[...])
    s_ref[...] = _dot(x, wst_ref[...]) + bs_ref[...]


_ntrans = pl.pallas_call(
    _ntrans_body,
    grid=(N_NODES // BN,),
    in_specs=[
        pl.BlockSpec((BN, HID), lambda i: (i, 0)),
        pl.BlockSpec((HID, HID), lambda i: (0, 0)),
        pl.BlockSpec((HID, HID), lambda i: (0, 0)),
        pl.BlockSpec((1, HID), lambda i: (0, 0)),
    ],
    out_specs=[
        pl.BlockSpec((BN, HID), lambda i: (i, 0)),
        pl.BlockSpec((BN, HID), lambda i: (i, 0)),
    ],
    out_shape=[
        jax.ShapeDtypeStruct((N_NODES, HID), f32),
        jax.ShapeDtypeStruct((N_NODES, HID), f32),
    ],
)


def _update_body(agg_ref, sp_ref, wa_ref, wst_ref, bs_ref, p_ref, s_ref):
    x = jnp.tanh(sp_ref[...] + agg_ref[0] + agg_ref[1])
    p_ref[...] = _dot(x, wa_ref[...])
    s_ref[...] = _dot(x, wst_ref[...]) + bs_ref[...]


_update = pl.pallas_call(
    _update_body,
    grid=(N_NODES // BN,),
    in_specs=[
        pl.BlockSpec((NC, BN, HID), lambda i: (0, i, 0)),
        pl.BlockSpec((BN, HID), lambda i: (i, 0)),
        pl.BlockSpec((HID, HID), lambda i: (0, 0)),
        pl.BlockSpec((HID, HID), lambda i: (0, 0)),
        pl.BlockSpec((1, HID), lambda i: (0, 0)),
    ],
    out_specs=[
        pl.BlockSpec((BN, HID), lambda i: (i, 0)),
        pl.BlockSpec((BN, HID), lambda i: (i, 0)),
    ],
    out_shape=[
        jax.ShapeDtypeStruct((N_NODES, HID), f32),
        jax.ShapeDtypeStruct((N_NODES, HID), f32),
    ],
)


def _final_body(agg_ref, sp_ref, ent_ref, out_ref):
    ns2 = jnp.tanh(sp_ref[...] + agg_ref[0] + agg_ref[1])
    v = jnp.mean(ent_ref[...], axis=0, keepdims=True)
    vn = jnp.sqrt(jnp.sum(v * v))
    vs = v / jnp.maximum(vn, 1e-12)
    rn = jnp.sqrt(jnp.sum(ns2 * ns2, axis=1, keepdims=True))
    nn = ns2 / jnp.maximum(rn, 1e-12)
    sims = _dot(nn, vs.T)
    m = jnp.max(sims)
    idx = lax.broadcasted_iota(jnp.int32, (N_NODES, 1), 0)
    best = jnp.min(jnp.where(sims >= m, idx, N_NODES))
    ctx = jnp.sum(jnp.where(idx == best, ns2, 0.0), axis=0, keepdims=True)
    out_ref[...] = jnp.broadcast_to(ctx, (BATCH, HID))


_final = pl.pallas_call(
    _final_body,
    in_specs=[
        pl.BlockSpec((NC, N_NODES, HID), lambda: (0, 0, 0)),
        pl.BlockSpec((N_NODES, HID), lambda: (0, 0)),
        pl.BlockSpec((BATCH, HID), lambda: (0, 0)),
    ],
    out_specs=pl.BlockSpec((BATCH, HID), lambda: (0, 0)),
    out_shape=jax.ShapeDtypeStruct((BATCH, HID), f32),
)


def kernel(entity_vec, node_reprs, edge_src, edge_dst, edge_rel, edge_conf,
           W_msg, b_msg, W_self, b_self):
    wa2 = 2.0 * W_msg[:, :HID].T
    wb2 = 2.0 * W_msg[:, HID:].T
    bm2 = 2.0 * b_msg.reshape(1, HID)
    wst = W_self.T
    bs = b_self.reshape(1, HID)
    conf_flat = edge_conf.reshape(N_EDGES)
    zeros = jnp.zeros((N_NODES, HID), f32)

    r2 = _relproj(edge_rel, wb2, bm2)
    p0, s0 = _ntrans(node_reprs, wa2, wst, bs)
    edge_pass = _edge_pass()
    agg0 = edge_pass(p0, r2, edge_src, edge_dst, conf_flat, zeros)
    p1, s1 = _update(agg0, s0, wa2, wst, bs)
    agg1 = edge_pass(p1, r2, edge_src, edge_dst, conf_flat, zeros)
    return _final(agg1, s1, entity_vec)


# speedup vs baseline: 2.1411x; 1.0521x over previous
import functools

import jax
import jax.numpy as jnp
from jax import lax
from jax.experimental import pallas as pl
from jax.experimental.pallas import tpu as pltpu
from jax.experimental.pallas import tpu_sc as plsc

N_NODES = 10000
N_EDGES = 320000
HID = 128
BATCH = 32

NC = 2
NS = 16
NW = NC * NS
EPW = N_EDGES // NW
CHUNK = 80
NCHUNK = EPW // CHUNK
ROWS_PT = 624
ROWS_TAIL = N_NODES - ROWS_PT * NS
LANES = 16
GROUPS = HID // LANES

f32 = jnp.float32


def _edge_body(p_hbm, r_hbm, src_hbm, dst_hbm, conf_hbm, zero_hbm, out_hbm,
               src0, dst0, conf0, r0, g0, src1, dst1, conf1, r1, g1,
               sdst0, sdst1, agg_sp,
               sem_l0, sem_g0, sem_l1, sem_g1, sem_s0, sem_s1):
    c = lax.axis_index("c")
    s = lax.axis_index("s")
    wid = c * NS + s
    base = wid * EPW

    pltpu.sync_copy(zero_hbm.at[pl.ds(s * ROWS_PT, ROWS_PT)],
                    agg_sp.at[pl.ds(s * ROWS_PT, ROWS_PT)])

    @pl.when(s == NS - 1)
    def _():
        pltpu.sync_copy(zero_hbm.at[pl.ds(ROWS_PT * NS, ROWS_TAIL)],
                        agg_sp.at[pl.ds(ROWS_PT * NS, ROWS_TAIL)])

    plsc.subcore_barrier()

    bufs = ((src0, dst0, conf0, r0, g0, sdst0, sem_l0, sem_g0, sem_s0),
            (src1, dst1, conf1, r1, g1, sdst1, sem_l1, sem_g1, sem_s1))

    def l_descs(j, b):
        src_b, dst_b, conf_b, r_b, _, _, sem_l, _, _ = bufs[b]
        off = base + j * CHUNK
        return (
            (src_hbm.at[pl.ds(off, CHUNK)], src_b, sem_l),
            (dst_hbm.at[pl.ds(off, CHUNK)], dst_b, sem_l),
            (conf_hbm.at[pl.ds(off, CHUNK)], conf_b, sem_l),
            (r_hbm.at[pl.ds(off, CHUNK)], r_b, sem_l),
        )

    def l_start(j, b):
        for d in l_descs(j, b):
            pltpu.async_copy(*d)

    def l_drain(j, b):
        for d in l_descs(j, b):
            pltpu.make_async_copy(*d).wait()

    def g_start(j, b):
        src_b, _, _, _, g_b, _, _, sem_g, _ = bufs[b]
        pltpu.async_copy(p_hbm.at[src_b], g_b, sem_g)

    def g_drain(j, b):
        src_b, _, _, _, g_b, _, _, sem_g, _ = bufs[b]
        pltpu.make_async_copy(p_hbm.at[src_b], g_b, sem_g).wait()

    def s_drain(b):
        _, _, _, _, g_b, sdst_b, _, _, sem_s = bufs[b]
        pltpu.make_async_copy(g_b, agg_sp.at[sdst_b], sem_s).wait()

    def compute(j, b):
        _, dst_b, conf_b, r_b, g_b, sdst_b, _, _, sem_s = bufs[b]

        for k in range(CHUNK // LANES):
            sl = pl.ds(k * LANES, LANES)
            sdst_b[sl] = dst_b[sl]

        def grp_body(g, carry):
            cf16 = conf_b[pl.ds(g * LANES, LANES)]
            cf2_16 = cf16 + cf16
            for i in range(LANES):
                cf = cf16[i]
                cf2 = cf2_16[i]
                e = g * LANES + i
                for h in range(GROUPS):
                    sl = pl.ds(h * LANES, LANES)
                    t = g_b[e, sl] + r_b[e, sl]
                    g_b[e, sl] = cf - cf2 / (jnp.exp(t) + 1.0)
            return carry

        lax.fori_loop(0, CHUNK // LANES, grp_body, 0, unroll=False)
        pltpu.async_copy(g_b, agg_sp.at[sdst_b], sem_s, add=True)

    l_start(0, 0)
    l_drain(0, 0)
    g_start(0, 0)
    l_start(1, 1)

    def pair_body(t, carry):
        j0 = 2 * t
        j1 = j0 + 1
        l_drain(j1, 1)

        @pl.when(t > 0)
        def _():
            s_drain(1)
        g_start(j1, 1)
        g_drain(j0, 0)
        compute(j0, 0)
        l_start(j0 + 2, 0)

        g_drain(j1, 1)
        compute(j1, 1)

        @pl.when(j0 + 3 < NCHUNK)
        def _():
            l_start(j0 + 3, 1)

        l_drain(j0 + 2, 0)
        s_drain(0)
        g_start(j0 + 2, 0)
        return carry

    lax.fori_loop(0, NCHUNK // 2, pair_body, 0, unroll=False)
    if NCHUNK % 2:
        g_drain(NCHUNK - 1, 0)
        compute(NCHUNK - 1, 0)
        s_drain(0)
        s_drain(1)
    else:
        s_drain(0)
        s_drain(1)

    plsc.subcore_barrier()
    pltpu.sync_copy(agg_sp.at[pl.ds(s * ROWS_PT, ROWS_PT)],
                    out_hbm.at[c, pl.ds(s * ROWS_PT, ROWS_PT)])

    @pl.when(s == NS - 1)
    def _():
        pltpu.sync_copy(agg_sp.at[pl.ds(ROWS_PT * NS, ROWS_TAIL)],
                        out_hbm.at[c, pl.ds(ROWS_PT * NS, ROWS_TAIL)])


@functools.cache
def _edge_pass():
    return functools.partial(
        pl.kernel,
        out_type=jax.ShapeDtypeStruct((NC, N_NODES, HID), f32),
        mesh=plsc.VectorSubcoreMesh(core_axis_name="c", subcore_axis_name="s",
                                    num_cores=NC, num_subcores=NS),
        scratch_types=[
            pltpu.VMEM((CHUNK,), jnp.int32),
            pltpu.VMEM((CHUNK,), jnp.int32),
            pltpu.VMEM((CHUNK,), f32),
            pltpu.VMEM((CHUNK, HID), f32),
            pltpu.VMEM((CHUNK, HID), f32),
            pltpu.VMEM((CHUNK,), jnp.int32),
            pltpu.VMEM((CHUNK,), jnp.int32),
            pltpu.VMEM((CHUNK,), f32),
            pltpu.VMEM((CHUNK, HID), f32),
            pltpu.VMEM((CHUNK, HID), f32),
            pltpu.VMEM((CHUNK,), jnp.int32),
            pltpu.VMEM((CHUNK,), jnp.int32),
            pltpu.VMEM_SHARED((N_NODES, HID), f32),
            pltpu.SemaphoreType.DMA,
            pltpu.SemaphoreType.DMA,
            pltpu.SemaphoreType.DMA,
            pltpu.SemaphoreType.DMA,
            pltpu.SemaphoreType.DMA,
            pltpu.SemaphoreType.DMA,
        ],
    )(_edge_body)


def _dot(x, w):
    return jnp.dot(x, w, preferred_element_type=f32,
                   precision=lax.Precision.HIGHEST)


def _relproj_body(rel_ref, wb_ref, b_ref, out_ref):
    out_ref[...] = _dot(rel_ref[...], wb_ref[...]) + b_ref[...]


BE = 2000

_relproj = pl.pallas_call(
    _relproj_body,
    grid=(N_EDGES // BE,),
    in_specs=[
        pl.BlockSpec((BE, HID), lambda i: (i, 0)),
        pl.BlockSpec((HID, HID), lambda i: (0, 0)),
        pl.BlockSpec((1, HID), lambda i: (0, 0)),
    ],
    out_specs=pl.BlockSpec((BE, HID), lambda i: (i, 0)),
    out_shape=jax.ShapeDtypeStruct((N_EDGES, HID), f32),
)

BN = 2000


def _ntrans_body(ns_ref, wa_ref, wst_ref, bs_ref, p_ref, s_ref):
    x = ns_ref[...]
    p_ref[...] = _dot(x, wa_ref[...])
    s_ref[...] = _dot(x, wst_ref[...]) + bs_ref[...]


_ntrans = pl.pallas_call(
    _ntrans_body,
    grid=(N_NODES // BN,),
    in_specs=[
        pl.BlockSpec((BN, HID), lambda i: (i, 0)),
        pl.BlockSpec((HID, HID), lambda i: (0, 0)),
        pl.BlockSpec((HID, HID), lambda i: (0, 0)),
        pl.BlockSpec((1, HID), lambda i: (0, 0)),
    ],
    out_specs=[
        pl.BlockSpec((BN, HID), lambda i: (i, 0)),
        pl.BlockSpec((BN, HID), lambda i: (i, 0)),
    ],
    out_shape=[
        jax.ShapeDtypeStruct((N_NODES, HID), f32),
        jax.ShapeDtypeStruct((N_NODES, HID), f32),
    ],
)


def _update_body(agg_ref, sp_ref, wa_ref, wst_ref, bs_ref, p_ref, s_ref):
    x = jnp.tanh(sp_ref[...] + agg_ref[0] + agg_ref[1])
    p_ref[...] = _dot(x, wa_ref[...])
    s_ref[...] = _dot(x, wst_ref[...]) + bs_ref[...]


_update = pl.pallas_call(
    _update_body,
    grid=(N_NODES // BN,),
    in_specs=[
        pl.BlockSpec((NC, BN, HID), lambda i: (0, i, 0)),
        pl.BlockSpec((BN, HID), lambda i: (i, 0)),
        pl.BlockSpec((HID, HID), lambda i: (0, 0)),
        pl.BlockSpec((HID, HID), lambda i: (0, 0)),
        pl.BlockSpec((1, HID), lambda i: (0, 0)),
    ],
    out_specs=[
        pl.BlockSpec((BN, HID), lambda i: (i, 0)),
        pl.BlockSpec((BN, HID), lambda i: (i, 0)),
    ],
    out_shape=[
        jax.ShapeDtypeStruct((N_NODES, HID), f32),
        jax.ShapeDtypeStruct((N_NODES, HID), f32),
    ],
)


def _final_body(agg_ref, sp_ref, ent_ref, out_ref):
    ns2 = jnp.tanh(sp_ref[...] + agg_ref[0] + agg_ref[1])
    v = jnp.mean(ent_ref[...], axis=0, keepdims=True)
    vn = jnp.sqrt(jnp.sum(v * v))
    vs = v / jnp.maximum(vn, 1e-12)
    rn = jnp.sqrt(jnp.sum(ns2 * ns2, axis=1, keepdims=True))
    nn = ns2 / jnp.maximum(rn, 1e-12)
    sims = _dot(nn, vs.T)
    m = jnp.max(sims)
    idx = lax.broadcasted_iota(jnp.int32, (N_NODES, 1), 0)
    best = jnp.min(jnp.where(sims >= m, idx, N_NODES))
    ctx = jnp.sum(jnp.where(idx == best, ns2, 0.0), axis=0, keepdims=True)
    out_ref[...] = jnp.broadcast_to(ctx, (BATCH, HID))


_final = pl.pallas_call(
    _final_body,
    in_specs=[
        pl.BlockSpec((NC, N_NODES, HID), lambda: (0, 0, 0)),
        pl.BlockSpec((N_NODES, HID), lambda: (0, 0)),
        pl.BlockSpec((BATCH, HID), lambda: (0, 0)),
    ],
    out_specs=pl.BlockSpec((BATCH, HID), lambda: (0, 0)),
    out_shape=jax.ShapeDtypeStruct((BATCH, HID), f32),
)


def kernel(entity_vec, node_reprs, edge_src, edge_dst, edge_rel, edge_conf,
           W_msg, b_msg, W_self, b_self):
    wa2 = 2.0 * W_msg[:, :HID].T
    wb2 = 2.0 * W_msg[:, HID:].T
    bm2 = 2.0 * b_msg.reshape(1, HID)
    wst = W_self.T
    bs = b_self.reshape(1, HID)
    conf_flat = edge_conf.reshape(N_EDGES)
    zeros = jnp.zeros((N_NODES, HID), f32)

    r2 = _relproj(edge_rel, wb2, bm2)
    p0, s0 = _ntrans(node_reprs, wa2, wst, bs)
    edge_pass = _edge_pass()
    agg0 = edge_pass(p0, r2, edge_src, edge_dst, conf_flat, zeros)
    p1, s1 = _update(agg0, s0, wa2, wst, bs)
    agg1 = edge_pass(p1, r2, edge_src, edge_dst, conf_flat, zeros)
    return _final(agg1, s1, entity_vec)
